# Initial kernel scaffold; baseline (speedup 1.0000x reference)
import jax
import jax.numpy as jnp
from jax.experimental import pallas as pl


def kernel(inputs, embeddings_weight):
    raise NotImplementedError("write your pallas kernel here")


# speedup vs baseline: 11.2798x; 11.2798x over previous
import jax
import jax.numpy as jnp
from jax.experimental import pallas as pl

_B = 4096
_K = 8192
_E = 32
_TB = 256
_TK = 2048


def _vq_onehot_kernel(x_ref, et_ref, out_ref):
    x = x_ref[:]
    nk = _K // _TK

    bmin = jnp.full((_TB, 1), jnp.inf, dtype=jnp.float32)
    bidx = jnp.zeros((_TB, 1), dtype=jnp.int32)
    for j in range(nk):
        et = et_ref[:, j * _TK:(j + 1) * _TK]
        s = jax.lax.dot_general(
            x, et, (((1,), (0,)), ((), ())), preferred_element_type=jnp.float32
        )
        e_sq = jnp.sum(et * et, axis=0, keepdims=True)
        d = e_sq - 2.0 * s
        lmin = jnp.min(d, axis=1, keepdims=True)
        iota = jax.lax.broadcasted_iota(jnp.int32, (_TB, _TK), 1)
        lidx = jnp.min(
            jnp.where(d == lmin, iota, _K), axis=1, keepdims=True
        ) + j * _TK
        upd = lmin < bmin
        bmin = jnp.where(upd, lmin, bmin)
        bidx = jnp.where(upd, lidx, bidx)

    for j in range(nk):
        iota = jax.lax.broadcasted_iota(jnp.int32, (_TB, _TK), 1) + j * _TK
        out_ref[:, j * _TK:(j + 1) * _TK] = (iota == bidx).astype(jnp.float32)


def kernel(inputs, embeddings_weight):
    b = inputs.shape[0]
    flat = inputs.reshape(b, _E)
    et = embeddings_weight.reshape(_K, _E).T
    return pl.pallas_call(
        _vq_onehot_kernel,
        grid=(b // _TB,),
        in_specs=[
            pl.BlockSpec((_TB, _E), lambda i: (i, 0)),
            pl.BlockSpec((_E, _K), lambda i: (0, 0)),
        ],
        out_specs=pl.BlockSpec((_TB, _K), lambda i: (i, 0)),
        out_shape=jax.ShapeDtypeStruct((b, _K), jnp.float32),
    )(flat, et)
